# baseline (speedup 1.0000x reference)
import functools

import jax
import jax.numpy as jnp
from jax import lax
from jax.experimental import pallas as pl
from jax.experimental.pallas import tpu as pltpu
from jax.experimental.pallas import tpu_sc as plsc

_LANES = 16
_NUM_TILES = 32
_VP = 1024
_NB = _VP // 256


def _counts_sc(slate):
    B, S = slate.shape
    per_tile = B // _NUM_TILES
    n_groups = per_tile // _LANES
    n_pairs = n_groups // 2

    mesh = plsc.VectorSubcoreMesh(core_axis_name="c", subcore_axis_name="s")
    nc = mesh.num_cores

    @functools.partial(
        pl.kernel,
        out_type=jax.ShapeDtypeStruct((B, 256), jnp.int32),
        mesh=mesh,
        compiler_params=pltpu.CompilerParams(needs_layout_passes=False),
        scratch_types=[
            pltpu.VMEM((_LANES, S), jnp.int32),
            pltpu.VMEM((_LANES, S), jnp.int32),
            pltpu.VMEM((_LANES, 256), jnp.int32),
            pltpu.VMEM((_LANES, 256), jnp.int32),
            pltpu.SemaphoreType.DMA,
            pltpu.SemaphoreType.DMA,
            pltpu.SemaphoreType.DMA,
            pltpu.SemaphoreType.DMA,
        ],
    )
    def k(slate_hbm, cp_hbm, sl0, sl1, cnt0, cnt1, si0, si1, so0, so1):
        wid = lax.axis_index("s") * nc + lax.axis_index("c")
        lane = lax.iota(jnp.int32, 16)
        one = jnp.ones((_LANES,), jnp.int32)
        zeros = jnp.zeros((_LANES,), jnp.int32)
        base = wid * per_tile

        def slate_src(g):
            return slate_hbm.at[pl.ds(base + g * _LANES, _LANES), :]

        def fetch(g, sl, si):
            pltpu.async_copy(slate_src(g), sl, si)

        def wait_fetch(g, sl, si):
            pltpu.make_async_copy(slate_src(0), sl, si).wait()

        def do_group(g, sl, cnt, so):
            def zr(rr, carry):
                for l in range(_LANES):
                    cnt[l, pl.ds(rr * _LANES, _LANES)] = zeros
                return carry

            lax.fori_loop(0, 256 // _LANES, zr, 0)
            for i in range(S):
                idx = plsc.load_gather(sl, [lane, jnp.full((_LANES,), i, jnp.int32)])
                col = jnp.right_shift(idx, 2)
                val = jnp.left_shift(one, jnp.left_shift(jnp.bitwise_and(idx, 3), 3))
                plsc.addupdate_scatter(cnt, [lane, col], val)
            pltpu.async_copy(
                cnt, cp_hbm.at[pl.ds(base + g * _LANES, _LANES), :], so
            )

        def drain_out(cnt, so):
            pltpu.make_async_copy(
                cnt, cp_hbm.at[pl.ds(0, _LANES), :], so
            ).wait()

        fetch(0, sl0, si0)

        def pair(h, carry):
            g0 = 2 * h
            fetch(g0 + 1, sl1, si1)
            wait_fetch(g0, sl0, si0)

            @pl.when(h > 0)
            def _():
                drain_out(cnt0, so0)

            do_group(g0, sl0, cnt0, so0)

            @pl.when(h < n_pairs - 1)
            def _():
                fetch(g0 + 2, sl0, si0)

            wait_fetch(g0 + 1, sl1, si1)

            @pl.when(h > 0)
            def _():
                drain_out(cnt1, so1)

            do_group(g0 + 1, sl1, cnt1, so1)
            return carry

        lax.fori_loop(0, n_pairs, pair, 0)
        drain_out(cnt0, so0)
        drain_out(cnt1, so1)

    return k(slate)


def _diversity_tc(c8_mat, sims_pad, S, blk):
    B = c8_mat.shape[0]
    denom = S * (S - 1)

    def body(m_ref, cp_ref, o_ref, madj_ref):
        @pl.when(pl.program_id(0) == 0)
        def _():
            ii = lax.broadcasted_iota(jnp.int32, (_VP, _VP), 0)
            jj = lax.broadcasted_iota(jnp.int32, (_VP, _VP), 1)
            mm = m_ref[...]
            dv = jnp.sum(jnp.where(ii == jj, mm, 0.0), axis=1, keepdims=True)
            madj_ref[...] = (mm - dv * (1.0 / S)).astype(jnp.bfloat16)

        cb = cp_ref[...].astype(jnp.bfloat16)
        z = jnp.dot(
            cb, madj_ref[...], preferred_element_type=jnp.float32
        )
        t = jnp.dot(
            z.astype(jnp.bfloat16) * cb, jnp.ones((_VP, 1), jnp.bfloat16),
            preferred_element_type=jnp.float32,
        )
        o_ref[...] = (t * (1.0 / denom)).reshape(blk)

    return pl.pallas_call(
        body,
        grid=(B // blk,),
        in_specs=[
            pl.BlockSpec((_VP, _VP), lambda j: (0, 0)),
            pl.BlockSpec((blk, _VP), lambda j: (j, 0)),
        ],
        out_specs=pl.BlockSpec((blk,), lambda j: (j,)),
        out_shape=jax.ShapeDtypeStruct((B,), jnp.float32),
        scratch_shapes=[pltpu.VMEM((_VP, _VP), jnp.bfloat16)],
    )(sims_pad, c8_mat)


def kernel(slate, item_item_similarities):
    B, S = slate.shape
    V = item_item_similarities.shape[0]
    sims_pad = jnp.pad(
        item_item_similarities, ((0, _VP - V), (0, _VP - V))
    )
    n_chunks = 4
    bc = B // n_chunks
    cps = [_counts_sc(slate[k * bc:(k + 1) * bc]) for k in range(n_chunks)]
    c8s = [
        jax.lax.bitcast_convert_type(cp, jnp.int8).reshape(bc, _VP)
        for cp in cps
    ]
    outs = [_diversity_tc(c8, sims_pad, S, 1024) for c8 in c8s]
    return jnp.concatenate(outs)


# speedup vs baseline: 3.1071x; 3.1071x over previous
import functools

import jax
import jax.numpy as jnp
from jax import lax
from jax.experimental import pallas as pl
from jax.experimental.pallas import tpu as pltpu
from jax.experimental.pallas import tpu_sc as plsc

_LANES = 16
_NUM_TILES = 32
_VP = 1024
_NB = _VP // 256


def _counts_sc(slate):
    B, S = slate.shape
    per_tile = B // _NUM_TILES
    n_groups = per_tile // _LANES
    n_pairs = n_groups // 2

    mesh = plsc.VectorSubcoreMesh(core_axis_name="c", subcore_axis_name="s")
    nc = mesh.num_cores

    @functools.partial(
        pl.kernel,
        out_type=jax.ShapeDtypeStruct((B, 256), jnp.int32),
        mesh=mesh,
        compiler_params=pltpu.CompilerParams(needs_layout_passes=False),
        scratch_types=[
            pltpu.VMEM((_LANES, S), jnp.int32),
            pltpu.VMEM((_LANES, S), jnp.int32),
            pltpu.VMEM((_LANES, 256), jnp.int32),
            pltpu.VMEM((_LANES, 256), jnp.int32),
            pltpu.SemaphoreType.DMA,
            pltpu.SemaphoreType.DMA,
            pltpu.SemaphoreType.DMA,
            pltpu.SemaphoreType.DMA,
        ],
    )
    def k(slate_hbm, cp_hbm, sl0, sl1, cnt0, cnt1, si0, si1, so0, so1):
        wid = lax.axis_index("s") * nc + lax.axis_index("c")
        lane = lax.iota(jnp.int32, 16)
        one = jnp.ones((_LANES,), jnp.int32)
        zeros = jnp.zeros((_LANES,), jnp.int32)
        base = wid * per_tile

        def slate_src(g):
            return slate_hbm.at[pl.ds(base + g * _LANES, _LANES), :]

        def fetch(g, sl, si):
            pltpu.async_copy(slate_src(g), sl, si)

        def wait_fetch(g, sl, si):
            pltpu.make_async_copy(slate_src(0), sl, si).wait()

        def do_group(g, sl, cnt, so):
            def zr(rr, carry):
                for l in range(_LANES):
                    cnt[l, pl.ds(rr * _LANES, _LANES)] = zeros
                return carry

            lax.fori_loop(0, 256 // _LANES, zr, 0)
            for i in range(S):
                idx = plsc.load_gather(sl, [lane, jnp.full((_LANES,), i, jnp.int32)])
                col = jnp.bitwise_and(idx, 255)
                val = jnp.left_shift(one, jnp.left_shift(jnp.right_shift(idx, 8), 3))
                plsc.addupdate_scatter(cnt, [lane, col], val)
            pltpu.async_copy(
                cnt, cp_hbm.at[pl.ds(base + g * _LANES, _LANES), :], so
            )

        def drain_out(cnt, so):
            pltpu.make_async_copy(
                cnt, cp_hbm.at[pl.ds(0, _LANES), :], so
            ).wait()

        fetch(0, sl0, si0)

        def pair(h, carry):
            g0 = 2 * h
            fetch(g0 + 1, sl1, si1)
            wait_fetch(g0, sl0, si0)

            @pl.when(h > 0)
            def _():
                drain_out(cnt0, so0)

            do_group(g0, sl0, cnt0, so0)

            @pl.when(h < n_pairs - 1)
            def _():
                fetch(g0 + 2, sl0, si0)

            wait_fetch(g0 + 1, sl1, si1)

            @pl.when(h > 0)
            def _():
                drain_out(cnt1, so1)

            do_group(g0 + 1, sl1, cnt1, so1)
            return carry

        lax.fori_loop(0, n_pairs, pair, 0)
        drain_out(cnt0, so0)
        drain_out(cnt1, so1)

    return k(slate)


def _diversity_tc(cp_mat, sims_pad, S, blk):
    B = cp_mat.shape[0]
    denom = S * (S - 1)

    def body(m_ref, cp_ref, o_ref, madj_ref):
        @pl.when(pl.program_id(0) == 0)
        def _():
            ii = lax.broadcasted_iota(jnp.int32, (_VP, _VP), 0)
            jj = lax.broadcasted_iota(jnp.int32, (_VP, _VP), 1)
            mm = m_ref[...]
            dv = jnp.sum(jnp.where(ii == jj, mm, 0.0), axis=1, keepdims=True)
            madj_ref[...] = (mm - dv * (1.0 / S)).astype(jnp.bfloat16)

        x = cp_ref[...]
        parts = [
            jnp.bitwise_and(jnp.right_shift(x, 8 * r), 255).astype(jnp.bfloat16)
            for r in range(_NB)
        ]
        cb = jnp.concatenate(parts, axis=1)
        z = jnp.dot(
            cb, madj_ref[...], preferred_element_type=jnp.float32
        )
        t = jnp.dot(
            z.astype(jnp.bfloat16) * cb, jnp.ones((_VP, 1), jnp.bfloat16),
            preferred_element_type=jnp.float32,
        )
        o_ref[...] = (t * (1.0 / denom)).reshape(blk)

    return pl.pallas_call(
        body,
        grid=(B // blk,),
        in_specs=[
            pl.BlockSpec((_VP, _VP), lambda j: (0, 0)),
            pl.BlockSpec((blk, 256), lambda j: (j, 0)),
        ],
        out_specs=pl.BlockSpec((blk,), lambda j: (j,)),
        out_shape=jax.ShapeDtypeStruct((B,), jnp.float32),
        scratch_shapes=[pltpu.VMEM((_VP, _VP), jnp.bfloat16)],
    )(sims_pad, cp_mat)


def kernel(slate, item_item_similarities):
    B, S = slate.shape
    V = item_item_similarities.shape[0]
    sims_pad = jnp.pad(
        item_item_similarities, ((0, _VP - V), (0, _VP - V))
    )
    n_chunks = 4
    bc = B // n_chunks
    cps = [_counts_sc(slate[k * bc:(k + 1) * bc]) for k in range(n_chunks)]
    outs = [_diversity_tc(cp, sims_pad, S, 1024) for cp in cps]
    return jnp.concatenate(outs)
